# Initial kernel scaffold; baseline (speedup 1.0000x reference)
import jax
import jax.numpy as jnp
from jax.experimental import pallas as pl


def kernel(x, pos_emb):
    raise NotImplementedError("write your pallas kernel here")


# speedup vs baseline: 2.4616x; 2.4616x over previous
import jax
import jax.numpy as jnp
from jax.experimental import pallas as pl


_SBLK = 256


def _add_tile(x_ref, pe_ref, o_ref):
    o_ref[...] = x_ref[...] + pe_ref[...]


def kernel(x, pos_emb):
    B, S, D = x.shape
    sblk = _SBLK if S % _SBLK == 0 else S
    grid = (S // sblk, B)
    return pl.pallas_call(
        _add_tile,
        grid=grid,
        in_specs=[
            pl.BlockSpec((1, sblk, D), lambda s, b: (b, s, 0)),
            pl.BlockSpec((sblk, D), lambda s, b: (s, 0)),
        ],
        out_specs=pl.BlockSpec((1, sblk, D), lambda s, b: (b, s, 0)),
        out_shape=jax.ShapeDtypeStruct((B, S, D), x.dtype),
    )(x, pos_emb)


# speedup vs baseline: 2.5629x; 1.0412x over previous
import jax
import jax.numpy as jnp
from jax.experimental import pallas as pl


_SBLK = 512


def _add_tile(x_ref, pe_ref, o_ref):
    o_ref[...] = x_ref[...] + pe_ref[...]


def kernel(x, pos_emb):
    B, S, D = x.shape
    sblk = _SBLK if S % _SBLK == 0 else S
    grid = (S // sblk, B)
    return pl.pallas_call(
        _add_tile,
        grid=grid,
        in_specs=[
            pl.BlockSpec((1, sblk, D), lambda s, b: (b, s, 0)),
            pl.BlockSpec((sblk, D), lambda s, b: (s, 0)),
        ],
        out_specs=pl.BlockSpec((1, sblk, D), lambda s, b: (b, s, 0)),
        out_shape=jax.ShapeDtypeStruct((B, S, D), x.dtype),
    )(x, pos_emb)
